# baseline (speedup 1.0000x reference)
import functools

import jax
import jax.numpy as jnp
from jax import lax
from jax.experimental import pallas as pl
from jax.experimental.pallas import tpu as pltpu
from jax.experimental.pallas import tpu_sc as plsc

N = 10000
E = 320000
D = 128
H = 128
G = 64
OUT = 2

NC = 2
NS = 16
NW = NC * NS
LANES = 16

C = 128
NCHUNK = 80
PAIRS = NCHUNK // 2
PH = 16
EPW = C * NCHUNK
E_PAD = NW * EPW
NA = 10240
RPT = NA // NS
ZR = 64


def _vec_mesh():
    return plsc.VectorSubcoreMesh(core_axis_name="c", subcore_axis_name="s")


def _deg_body(dst_hbm, deg_hbm, ones_v, idx_v, zsrc_v, deg_sp, sem):
    c = lax.axis_index("c")
    s = lax.axis_index("s")
    wid = c * NS + s

    @pl.loop(0, C, step=LANES)
    def _(i):
        ones_v[pl.ds(i, LANES)] = jnp.full((LANES,), 1.0, jnp.float32)

    @pl.loop(0, RPT, step=LANES)
    def _(i):
        zsrc_v[pl.ds(i, LANES)] = jnp.zeros((LANES,), jnp.float32)

    pltpu.sync_copy(zsrc_v, deg_sp.at[pl.ds(s * RPT, RPT)])
    plsc.subcore_barrier()

    @pl.loop(0, NCHUNK // PH)
    def _(p):
        pltpu.sync_copy(dst_hbm.at[pl.ds(wid * NCHUNK + p * PH, PH)], idx_v)
        dd = [pltpu.async_copy(ones_v, deg_sp.at[idx_v.at[j]], sem, add=True)
              for j in range(PH)]
        for d in dd:
            d.wait()

    plsc.subcore_barrier()
    pltpu.sync_copy(deg_sp.at[pl.ds(s * RPT, RPT)],
                    deg_hbm.at[pl.ds(c * NA + s * RPT, RPT)])


def _deg_pass(dstp2d):
    k = pl.kernel(
        _deg_body,
        out_type=jax.ShapeDtypeStruct((NC * NA,), jnp.float32),
        mesh=_vec_mesh(),
        scratch_types=[
            pltpu.VMEM((C,), jnp.float32),
            pltpu.VMEM((PH, C), jnp.int32),
            pltpu.VMEM((RPT,), jnp.float32),
            pltpu.VMEM_SHARED((NA,), jnp.float32),
            pltpu.SemaphoreType.DMA,
        ],
    )
    return k(dstp2d)


def _edge_body(h_hbm, src_hbm, dst_hbm, acc_hbm,
               ia0, id0, ia1, id1, r0, r1, acc_sp, g0, g1, s0, s1, isem):
    c = lax.axis_index("c")
    s = lax.axis_index("s")
    wid = c * NS + s
    rows = [r0, r1]
    gsem = [g0, g1]
    ssem = [s0, s1]
    islabs = [(ia0, id0), (ia1, id1)]
    NPH = NCHUNK // PH
    base = wid * NCHUNK

    @pl.loop(0, C)
    def _(i):
        @pl.loop(0, D, step=LANES)
        def _(j):
            r0[i, pl.ds(j, LANES)] = jnp.zeros((LANES,), jnp.float32)

    @pl.loop(0, RPT // C)
    def _(k):
        pltpu.sync_copy(r0, acc_sp.at[pl.ds(s * RPT + k * C, C)])

    plsc.subcore_barrier()

    def idx_refs(j):
        p = j // PH
        ia, idd = islabs[p % 2]
        return ia.at[j % PH], idd.at[j % PH]

    def load_slab(p):
        ia, idd = islabs[p % 2]
        return (pltpu.async_copy(src_hbm.at[pl.ds(base + p * PH, PH)], ia,
                                 isem),
                pltpu.async_copy(dst_hbm.at[pl.ds(base + p * PH, PH)], idd,
                                 isem))

    sl = [None] * (NPH + 1)
    pltpu.sync_copy(src_hbm.at[pl.ds(base, PH)], ia0)
    pltpu.sync_copy(dst_hbm.at[pl.ds(base, PH)], id0)
    if NPH > 1:
        sl[1] = load_slab(1)

    g = [None] * NCHUNK
    sc = [None] * NCHUNK
    ia_, _ = idx_refs(0)
    g[0] = pltpu.async_copy(h_hbm.at[ia_], rows[0], gsem[0])
    for j in range(NCHUNK):
        b = j % 2
        g[j].wait()
        if j >= 1:
            sc[j - 1].wait()
        if j % PH == 1 and j >= PH and j // PH + 1 < NPH:
            sl[j // PH + 1] = load_slab(j // PH + 1)
        jn = j + 1
        if jn < NCHUNK:
            if jn % PH == 0:
                for d in sl[jn // PH]:
                    d.wait()
            ian, _ = idx_refs(jn)
            g[jn] = pltpu.async_copy(h_hbm.at[ian], rows[1 - b], gsem[1 - b])
        _, id_ = idx_refs(j)
        sc[j] = pltpu.async_copy(rows[b], acc_sp.at[id_], ssem[b], add=True)
    sc[NCHUNK - 1].wait()

    plsc.subcore_barrier()
    pltpu.sync_copy(acc_sp.at[pl.ds(s * RPT, RPT)],
                    acc_hbm.at[pl.ds(c * NA + s * RPT, RPT)])


def _edge_pass(h, srcp2d, dstp2d):
    k = pl.kernel(
        _edge_body,
        out_type=jax.ShapeDtypeStruct((NC * NA, D), jnp.float32),
        mesh=_vec_mesh(),
        scratch_types=[
            pltpu.VMEM((PH, C), jnp.int32),
            pltpu.VMEM((PH, C), jnp.int32),
            pltpu.VMEM((PH, C), jnp.int32),
            pltpu.VMEM((PH, C), jnp.int32),
            pltpu.VMEM((C, D), jnp.float32),
            pltpu.VMEM((C, D), jnp.float32),
            pltpu.VMEM_SHARED((NA, D), jnp.float32),
            pltpu.SemaphoreType.DMA,
            pltpu.SemaphoreType.DMA,
            pltpu.SemaphoreType.DMA,
            pltpu.SemaphoreType.DMA,
            pltpu.SemaphoreType.DMA,
        ],
    )
    return k(h, srcp2d, dstp2d)


def _dinv_from_deg(deg_ref):
    degsum = deg_ref[0:NA] + deg_ref[NA:2 * NA] + 1.0
    return lax.rsqrt(degsum)[:, None]


def _stage_a_body(x_ref, w1_ref, deg_ref, h1p_ref):
    dinv = _dinv_from_deg(deg_ref)
    h1 = jnp.dot(x_ref[...], w1_ref[...], preferred_element_type=jnp.float32)
    h1p_ref[0:N, :] = h1 * dinv[0:N, :]
    h1p_ref[N:NA, :] = jnp.zeros((NA - N, D), jnp.float32)


def _stage_a(x, W1, deg):
    return pl.pallas_call(
        _stage_a_body,
        out_shape=jax.ShapeDtypeStruct((NA, D), jnp.float32),
    )(x, W1, deg)


def _stage_b_body(acc_ref, h1p_ref, deg_ref, w2_ref, b1_ref, h2p_ref):
    dinv = _dinv_from_deg(deg_ref)
    accsum = acc_ref[0:NA, :] + acc_ref[NA:2 * NA, :]
    o1 = jnp.maximum((accsum + h1p_ref[...]) * dinv + b1_ref[...][None, :], 0.0)
    h2 = jnp.dot(o1, w2_ref[...], preferred_element_type=jnp.float32)
    h2p_ref[...] = h2 * dinv


def _stage_b(acc1, h1p, deg, W2, b1):
    return pl.pallas_call(
        _stage_b_body,
        out_shape=jax.ShapeDtypeStruct((NA, D), jnp.float32),
    )(acc1, h1p, deg, W2, b1)


def _stage_c_body(acc_ref, h2p_ref, deg_ref, b2_ref, wc_ref, bc_ref,
                  batch_ref, out_ref):
    dinv = _dinv_from_deg(deg_ref)
    accsum = acc_ref[0:NA, :] + acc_ref[NA:2 * NA, :]
    o2 = jnp.maximum((accsum + h2p_ref[...]) * dinv + b2_ref[...][None, :], 0.0)
    gids = lax.broadcasted_iota(jnp.int32, (G, NA), 0)
    oh = (gids == batch_ref[...][None, :]).astype(jnp.float32)
    sums = jnp.dot(oh, o2, preferred_element_type=jnp.float32)
    counts = jnp.sum(oh, axis=1)[:, None]
    pooled = sums / jnp.maximum(counts, 1.0)
    out_ref[...] = (jnp.dot(pooled, wc_ref[...],
                            preferred_element_type=jnp.float32)
                    + bc_ref[...][None, :])


def _stage_c(acc2, h2p, deg, b2, Wc, bc, batchp):
    return pl.pallas_call(
        _stage_c_body,
        out_shape=jax.ShapeDtypeStruct((G, OUT), jnp.float32),
    )(acc2, h2p, deg, b2, Wc, bc, batchp)


def kernel(x, edge_index, batch, W1, b1, W2, b2, Wc, bc):
    src, dst = edge_index[0], edge_index[1]
    pad = E_PAD - E
    padi = jnp.arange(pad, dtype=jnp.int32)
    srcp = jnp.concatenate([src, (padi * 997) % N]).reshape(NW * NCHUNK, C)
    dstp = jnp.concatenate([dst, N + (padi % (NA - N))]).reshape(NW * NCHUNK, C)
    batchp = jnp.concatenate([batch, jnp.full((NA - N,), G, jnp.int32)])

    deg = _deg_pass(dstp)
    h1p = _stage_a(x, W1, deg)
    acc1 = _edge_pass(h1p, srcp, dstp)
    h2p = _stage_b(acc1, h1p, deg, W2, b1)
    acc2 = _edge_pass(h2p, srcp, dstp)
    return _stage_c(acc2, h2p, deg, b2, Wc, bc, batchp)


# speedup vs baseline: 1.0053x; 1.0053x over previous
import jax
import jax.numpy as jnp
from jax import lax
from jax.experimental import pallas as pl
from jax.experimental.pallas import tpu as pltpu
from jax.experimental.pallas import tpu_sc as plsc

N = 10000
E = 320000
D = 128
H = 128
G = 64
OUT = 2

NC = 2
NS = 16
NW = NC * NS
LANES = 16

C = 128
NCHUNK = 80
PH = 16
EPW = C * NCHUNK
E_PAD = NW * EPW
NA = 10240
RPT = NA // NS


def _vec_mesh():
    return plsc.VectorSubcoreMesh(core_axis_name="c", subcore_axis_name="s")


def _deg_body(dst_hbm, deg_hbm, ones_v, idx_v, zsrc_v, deg_sp, sem):
    c = lax.axis_index("c")
    s = lax.axis_index("s")
    wid = c * NS + s

    @pl.loop(0, C, step=LANES)
    def _(i):
        ones_v[pl.ds(i, LANES)] = jnp.full((LANES,), 1.0, jnp.float32)

    @pl.loop(0, RPT, step=LANES)
    def _(i):
        zsrc_v[pl.ds(i, LANES)] = jnp.zeros((LANES,), jnp.float32)

    pltpu.sync_copy(zsrc_v, deg_sp.at[pl.ds(s * RPT, RPT)])
    plsc.subcore_barrier()

    @pl.loop(0, NCHUNK // PH)
    def _(p):
        pltpu.sync_copy(dst_hbm.at[pl.ds(wid * NCHUNK + p * PH, PH)], idx_v)
        dd = [pltpu.async_copy(ones_v, deg_sp.at[idx_v.at[j]], sem, add=True)
              for j in range(PH)]
        for d in dd:
            d.wait()

    plsc.subcore_barrier()
    pltpu.sync_copy(deg_sp.at[pl.ds(s * RPT, RPT)],
                    deg_hbm.at[pl.ds(c * NA + s * RPT, RPT)])


def _deg_pass(dstp2d):
    k = pl.kernel(
        _deg_body,
        out_type=jax.ShapeDtypeStruct((NC * NA,), jnp.float32),
        mesh=_vec_mesh(),
        scratch_types=[
            pltpu.VMEM((C,), jnp.float32),
            pltpu.VMEM((PH, C), jnp.int32),
            pltpu.VMEM((RPT,), jnp.float32),
            pltpu.VMEM_SHARED((NA,), jnp.float32),
            pltpu.SemaphoreType.DMA,
        ],
    )
    return k(dstp2d)


def _edge_body(h_hbm, src_hbm, dst_hbm, acc_hbm,
               ia0, id0, ia1, id1, r0, r1, acc_sp, g0, g1, s0, s1, isem):
    c = lax.axis_index("c")
    s = lax.axis_index("s")
    wid = c * NS + s
    rows = [r0, r1]
    gsem = [g0, g1]
    ssem = [s0, s1]
    islabs = [(ia0, id0), (ia1, id1)]
    NPH = NCHUNK // PH
    base = wid * NCHUNK

    @pl.loop(0, C)
    def _(i):
        @pl.loop(0, D, step=LANES)
        def _(j):
            r0[i, pl.ds(j, LANES)] = jnp.zeros((LANES,), jnp.float32)

    @pl.loop(0, RPT // C)
    def _(k):
        pltpu.sync_copy(r0, acc_sp.at[pl.ds(s * RPT + k * C, C)])

    plsc.subcore_barrier()

    def idx_refs(j):
        p = j // PH
        ia, idd = islabs[p % 2]
        return ia.at[j % PH], idd.at[j % PH]

    def load_slab(p):
        ia, idd = islabs[p % 2]
        return (pltpu.async_copy(src_hbm.at[pl.ds(base + p * PH, PH)], ia,
                                 isem),
                pltpu.async_copy(dst_hbm.at[pl.ds(base + p * PH, PH)], idd,
                                 isem))

    sl = [None] * (NPH + 1)
    pltpu.sync_copy(src_hbm.at[pl.ds(base, PH)], ia0)
    pltpu.sync_copy(dst_hbm.at[pl.ds(base, PH)], id0)
    if NPH > 1:
        sl[1] = load_slab(1)

    g = [None] * NCHUNK
    sc = [None] * NCHUNK
    ia_, _ = idx_refs(0)
    g[0] = pltpu.async_copy(h_hbm.at[ia_], rows[0], gsem[0])
    for j in range(NCHUNK):
        b = j % 2
        g[j].wait()
        if j >= 1:
            sc[j - 1].wait()
        if j % PH == 1 and j >= PH and j // PH + 1 < NPH:
            sl[j // PH + 1] = load_slab(j // PH + 1)
        jn = j + 1
        if jn < NCHUNK:
            if jn % PH == 0:
                for d in sl[jn // PH]:
                    d.wait()
            ian, _ = idx_refs(jn)
            g[jn] = pltpu.async_copy(h_hbm.at[ian], rows[1 - b], gsem[1 - b])
        _, id_ = idx_refs(j)
        sc[j] = pltpu.async_copy(rows[b], acc_sp.at[id_], ssem[b], add=True)
    sc[NCHUNK - 1].wait()

    plsc.subcore_barrier()
    pltpu.sync_copy(acc_sp.at[pl.ds(s * RPT, RPT)],
                    acc_hbm.at[pl.ds(c * NA + s * RPT, RPT)])


def _edge_pass(h, srcp2d, dstp2d):
    k = pl.kernel(
        _edge_body,
        out_type=jax.ShapeDtypeStruct((NC * NA, D), jnp.float32),
        mesh=_vec_mesh(),
        scratch_types=[
            pltpu.VMEM((PH, C), jnp.int32),
            pltpu.VMEM((PH, C), jnp.int32),
            pltpu.VMEM((PH, C), jnp.int32),
            pltpu.VMEM((PH, C), jnp.int32),
            pltpu.VMEM((C, D), jnp.float32),
            pltpu.VMEM((C, D), jnp.float32),
            pltpu.VMEM_SHARED((NA, D), jnp.float32),
            pltpu.SemaphoreType.DMA,
            pltpu.SemaphoreType.DMA,
            pltpu.SemaphoreType.DMA,
            pltpu.SemaphoreType.DMA,
            pltpu.SemaphoreType.DMA,
        ],
    )
    return k(h, srcp2d, dstp2d)


def _dinv_from_deg(deg_ref):
    degsum = deg_ref[0:NA] + deg_ref[NA:2 * NA] + 1.0
    return lax.rsqrt(degsum)[:, None]


def _stage_a_body(x_ref, w1_ref, deg_ref, h1p_ref):
    dinv = _dinv_from_deg(deg_ref)
    h1 = jnp.dot(x_ref[...], w1_ref[...], preferred_element_type=jnp.float32)
    h1p_ref[0:N, :] = h1 * dinv[0:N, :]
    h1p_ref[N:NA, :] = jnp.zeros((NA - N, D), jnp.float32)


def _stage_a(x, W1, deg):
    return pl.pallas_call(
        _stage_a_body,
        out_shape=jax.ShapeDtypeStruct((NA, D), jnp.float32),
    )(x, W1, deg)


def _stage_b_body(acc_ref, h1p_ref, deg_ref, w2_ref, b1_ref, h2p_ref):
    dinv = _dinv_from_deg(deg_ref)
    accsum = acc_ref[0:NA, :] + acc_ref[NA:2 * NA, :]
    o1 = jnp.maximum((accsum + h1p_ref[...]) * dinv + b1_ref[...][None, :], 0.0)
    h2 = jnp.dot(o1, w2_ref[...], preferred_element_type=jnp.float32)
    h2p_ref[...] = h2 * dinv


def _stage_b(acc1, h1p, deg, W2, b1):
    return pl.pallas_call(
        _stage_b_body,
        out_shape=jax.ShapeDtypeStruct((NA, D), jnp.float32),
    )(acc1, h1p, deg, W2, b1)


def _stage_c_body(acc_ref, h2p_ref, deg_ref, b2_ref, wc_ref, bc_ref,
                  batch_ref, out_ref):
    dinv = _dinv_from_deg(deg_ref)
    accsum = acc_ref[0:NA, :] + acc_ref[NA:2 * NA, :]
    o2 = jnp.maximum((accsum + h2p_ref[...]) * dinv + b2_ref[...][None, :], 0.0)
    gids = lax.broadcasted_iota(jnp.int32, (G, NA), 0)
    oh = (gids == batch_ref[...][None, :]).astype(jnp.float32)
    sums = jnp.dot(oh, o2, preferred_element_type=jnp.float32)
    counts = jnp.sum(oh, axis=1)[:, None]
    pooled = sums / jnp.maximum(counts, 1.0)
    out_ref[...] = (jnp.dot(pooled, wc_ref[...],
                            preferred_element_type=jnp.float32)
                    + bc_ref[...][None, :])


def _stage_c(acc2, h2p, deg, b2, Wc, bc, batchp):
    return pl.pallas_call(
        _stage_c_body,
        out_shape=jax.ShapeDtypeStruct((G, OUT), jnp.float32),
    )(acc2, h2p, deg, b2, Wc, bc, batchp)


def kernel(x, edge_index, batch, W1, b1, W2, b2, Wc, bc):
    src, dst = edge_index[0], edge_index[1]
    pad = E_PAD - E
    padi = jnp.arange(pad, dtype=jnp.int32)
    srcp = jnp.concatenate([src, (padi * 997) % N]).reshape(NW * NCHUNK, C)
    dstp = jnp.concatenate([dst, N + (padi % (NA - N))]).reshape(NW * NCHUNK, C)
    batchp = jnp.concatenate([batch, jnp.full((NA - N,), G, jnp.int32)])

    deg = _deg_pass(dstp)
    h1p = _stage_a(x, W1, deg)
    acc1 = _edge_pass(h1p, srcp, dstp)
    h2p = _stage_b(acc1, h1p, deg, W2, b1)
    acc2 = _edge_pass(h2p, srcp, dstp)
    return _stage_c(acc2, h2p, deg, b2, Wc, bc, batchp)
